# baseline (speedup 1.0000x reference)
import functools

import jax
import jax.numpy as jnp
from jax import lax
from jax.experimental import pallas as pl
from jax.experimental.pallas import tpu as pltpu
from jax.experimental.pallas import tpu_sc as plsc

_NC = 25
_B = 1024
_S = 1024
_IH = _B // 8
_JH = _S // 128
_NCHUNK = _IH * _JH
_NW = 32
_CPW = _NCHUNK // _NW
_PAIR = 2
_NB = _CPW // _PAIR
_W = _PAIR * 1024
_OUTW = _NC * _W


def _sc_call(idx_flat):
    mesh = plsc.VectorSubcoreMesh(core_axis_name="c", subcore_axis_name="s")

    @functools.partial(
        pl.kernel,
        mesh=mesh,
        compiler_params=pltpu.CompilerParams(needs_layout_passes=False),
        out_type=jax.ShapeDtypeStruct((_NC, _NCHUNK, 8, 128), jnp.float32),
        scratch_types=[
            pltpu.VMEM((_W,), jnp.int32),
            pltpu.VMEM((_W,), jnp.int32),
            pltpu.VMEM((_NC, _PAIR, 8, 128), jnp.float32),
            pltpu.VMEM((_NC, _PAIR, 8, 128), jnp.float32),
            pltpu.VMEM((_W,), jnp.int32),
            pltpu.VMEM((_W,), jnp.int32),
            pltpu.SemaphoreType.DMA,
            pltpu.SemaphoreType.DMA,
            pltpu.SemaphoreType.DMA,
            pltpu.SemaphoreType.DMA,
        ],
    )
    def k(idx_hbm, out_hbm, i0, i1, o0, o1, f0, f1, si0, si1, so0, so1):
        wid = lax.axis_index("s") * 2 + lax.axis_index("c")
        base_chunk = wid * _CPW
        zeros16f = jnp.zeros((16,), jnp.float32)
        ones16f = jnp.ones((16,), jnp.float32)
        zeros16i = jnp.zeros((16,), jnp.int32)
        iota16 = lax.iota(jnp.int32, 16)

        idx_v = (i0, i1)
        out_v = (o0, o1)
        off_v = (f0, f1)
        isem = (si0, si1)
        osem = (so0, so1)

        def zinit(t, _):
            c = t // (_PAIR * 8 * 8)
            rem = t - c * (_PAIR * 8 * 8)
            ch = rem // 64
            rem2 = rem - ch * 64
            r = rem2 // 8
            l = (rem2 - r * 8) * 16
            o0[c, ch, r, pl.ds(l, 16)] = zeros16f
            o1[c, ch, r, pl.ds(l, 16)] = zeros16f
            return 0

        lax.fori_loop(0, _NC * _PAIR * 8 * 8, zinit, 0)

        def cinit(g, _):
            f0[pl.ds(g * 16, 16)] = zeros16i
            f1[pl.ds(g * 16, 16)] = zeros16i
            return 0

        lax.fori_loop(0, _W // 16, cinit, 0)

        for b in range(2):
            pltpu.async_copy(
                idx_hbm.at[pl.ds((base_chunk + b * _PAIR) * 1024, _W)],
                idx_v[b], isem[b],
            )

        def run_batch(o, b):
            p = o * 2 + b
            chunk0 = base_chunk + p * _PAIR
            pltpu.make_async_copy(
                idx_hbm.at[pl.ds(chunk0 * 1024, _W)], idx_v[b], isem[b]
            ).wait()

            @pl.when(o >= 1)
            def _():
                pltpu.make_async_copy(
                    out_hbm.at[:, pl.ds(0, _PAIR)], out_v[b], osem[b]
                ).wait()

            def group(g, _):
                base = g * 16
                idx16 = idx_v[b][pl.ds(base, 16)]
                old16 = off_v[b][pl.ds(base, 16)]
                ch16 = jnp.broadcast_to(base // 1024, (16,)).astype(jnp.int32)
                r16 = jnp.broadcast_to((base % 1024) // 128, (16,)).astype(jnp.int32)
                l16 = (base % 128) + iota16
                plsc.store_scatter(out_v[b], [old16, ch16, r16, l16], zeros16f)
                off_v[b][pl.ds(base, 16)] = idx16
                plsc.store_scatter(out_v[b], [idx16, ch16, r16, l16], ones16f)
                return 0

            lax.fori_loop(0, _W // 16, group, 0, unroll=4)

            @pl.when(o < _NB // 2 - 1)
            def _():
                pltpu.async_copy(
                    idx_hbm.at[pl.ds((chunk0 + 2 * _PAIR) * 1024, _W)],
                    idx_v[b], isem[b],
                )

            pltpu.async_copy(
                out_v[b],
                out_hbm.at[:, pl.ds(chunk0, _PAIR)],
                osem[b],
            )
            return 0

        def outer(o, _):
            run_batch(o, 0)
            run_batch(o, 1)
            return 0

        lax.fori_loop(0, _NB // 2, outer, 0)

        for b in range(2):
            pltpu.make_async_copy(
                out_hbm.at[:, pl.ds(0, _PAIR)], out_v[b], osem[b]
            ).wait()

    return k(idx_flat)


def kernel(inputs):
    t = (
        inputs.reshape(_IH, 8, _JH, 128)
        .transpose(0, 2, 1, 3)
        .reshape(_NCHUNK * 1024)
    )
    y = _sc_call(t)
    y5 = y.reshape(_NC, _IH, _JH, 8, 128)
    return y5.transpose(1, 3, 2, 4, 0).reshape(_B, _S, _NC)


# speedup vs baseline: 1.0875x; 1.0875x over previous
import functools

import jax
import jax.numpy as jnp
from jax import lax
from jax.experimental import pallas as pl
from jax.experimental.pallas import tpu as pltpu
from jax.experimental.pallas import tpu_sc as plsc

_NC = 25
_B = 1024
_S = 1024
_IH = _B // 8
_JH = _S // 128
_NCHUNK = _IH * _JH
_NW = 32
_CPW = _NCHUNK // _NW
_W = 1024
_NBUF = 4


def _sc_call(idx_flat):
    mesh = plsc.VectorSubcoreMesh(core_axis_name="c", subcore_axis_name="s")

    @functools.partial(
        pl.kernel,
        mesh=mesh,
        compiler_params=pltpu.CompilerParams(needs_layout_passes=False),
        out_type=jax.ShapeDtypeStruct((_NC, _NCHUNK, 8, 128), jnp.float32),
        scratch_types=[
            pltpu.VMEM((_NBUF, _W), jnp.int32),
            pltpu.VMEM((_NBUF, _NC, 1, 8, 128), jnp.float32),
            pltpu.VMEM((_NBUF, _W), jnp.int32),
            pltpu.SemaphoreType.DMA,
            pltpu.SemaphoreType.DMA,
            pltpu.SemaphoreType.DMA,
            pltpu.SemaphoreType.DMA,
            pltpu.SemaphoreType.DMA,
            pltpu.SemaphoreType.DMA,
            pltpu.SemaphoreType.DMA,
            pltpu.SemaphoreType.DMA,
        ],
    )
    def k(idx_hbm, out_hbm, idxs, outs, olds, si0, si1, si2, si3, so0, so1, so2, so3):
        wid = lax.axis_index("s") * 2 + lax.axis_index("c")
        base_chunk = wid * _CPW
        zeros16f = jnp.zeros((16,), jnp.float32)
        ones16f = jnp.ones((16,), jnp.float32)
        zeros16i = jnp.zeros((16,), jnp.int32)
        iota16 = lax.iota(jnp.int32, 16)
        isem = (si0, si1, si2, si3)
        osem = (so0, so1, so2, so3)

        def zinit(t, _):
            c = t // (8 * 8)
            rem = t - c * 64
            r = rem // 8
            l = (rem - r * 8) * 16
            for b in range(_NBUF):
                outs[b, c, 0, r, pl.ds(l, 16)] = zeros16f
            return 0

        lax.fori_loop(0, _NC * 64, zinit, 0)

        def cinit(g, _):
            for b in range(_NBUF):
                olds[b, pl.ds(g * 16, 16)] = zeros16i
            return 0

        lax.fori_loop(0, _W // 16, cinit, 0)

        for b in range(_NBUF):
            pltpu.async_copy(
                idx_hbm.at[pl.ds((base_chunk + b) * 1024, _W)],
                idxs.at[b], isem[b],
            )

        def run_batch(o, b):
            p = o * _NBUF + b
            chunk = base_chunk + p
            pltpu.make_async_copy(
                idx_hbm.at[pl.ds(chunk * 1024, _W)], idxs.at[b], isem[b]
            ).wait()

            @pl.when(o >= 1)
            def _():
                pltpu.make_async_copy(
                    out_hbm.at[:, pl.ds(0, 1)], outs.at[b], osem[b]
                ).wait()

            def group(g, _):
                base = g * 16
                idx16 = idxs[b, pl.ds(base, 16)]
                old16 = olds[b, pl.ds(base, 16)]
                ch16 = zeros16i
                r16 = jnp.broadcast_to(base // 128, (16,)).astype(jnp.int32)
                l16 = (base % 128) + iota16
                plsc.store_scatter(outs.at[b], [old16, ch16, r16, l16], zeros16f)
                olds[b, pl.ds(base, 16)] = idx16
                plsc.store_scatter(outs.at[b], [idx16, ch16, r16, l16], ones16f)
                return 0

            lax.fori_loop(0, _W // 16, group, 0, unroll=4)

            @pl.when(o < _CPW // _NBUF - 1)
            def _():
                pltpu.async_copy(
                    idx_hbm.at[pl.ds((chunk + _NBUF) * 1024, _W)],
                    idxs.at[b], isem[b],
                )

            pltpu.async_copy(
                outs.at[b],
                out_hbm.at[:, pl.ds(chunk, 1)],
                osem[b],
            )
            return 0

        def outer(o, _):
            for b in range(_NBUF):
                run_batch(o, b)
            return 0

        lax.fori_loop(0, _CPW // _NBUF, outer, 0)

        for b in range(_NBUF):
            pltpu.make_async_copy(
                out_hbm.at[:, pl.ds(0, 1)], outs.at[b], osem[b]
            ).wait()

    return k(idx_flat)


def kernel(inputs):
    t = (
        inputs.reshape(_IH, 8, _JH, 128)
        .transpose(0, 2, 1, 3)
        .reshape(_NCHUNK * 1024)
    )
    y = _sc_call(t)
    y5 = y.reshape(_NC, _IH, _JH, 8, 128)
    return y5.transpose(1, 3, 2, 4, 0).reshape(_B, _S, _NC)


# speedup vs baseline: 1.1195x; 1.0295x over previous
import functools

import jax
import jax.numpy as jnp
from jax import lax
from jax.experimental import pallas as pl
from jax.experimental.pallas import tpu as pltpu
from jax.experimental.pallas import tpu_sc as plsc

_NC = 25
_B = 1024
_S = 1024
_IH = _B // 8
_JH = _S // 128
_NCHUNK = _IH * _JH
_NW = 32
_CPW = _NCHUNK // _NW
_W = 1024
_OUTW = _NC * _W
_NBUF = 4
_PLANE = _NCHUNK * 1024


def _sc_call(idx_flat):
    mesh = plsc.VectorSubcoreMesh(core_axis_name="c", subcore_axis_name="s")

    @functools.partial(
        pl.kernel,
        mesh=mesh,
        compiler_params=pltpu.CompilerParams(needs_layout_passes=False),
        out_type=jax.ShapeDtypeStruct((_NC * _PLANE,), jnp.float32),
        scratch_types=[
            pltpu.VMEM((_NBUF, _W), jnp.int32),
            pltpu.VMEM((_OUTW,), jnp.float32),
            pltpu.VMEM((_OUTW,), jnp.float32),
            pltpu.VMEM((_OUTW,), jnp.float32),
            pltpu.VMEM((_OUTW,), jnp.float32),
            pltpu.VMEM((_NBUF, _W), jnp.int32),
            pltpu.SemaphoreType.DMA,
            pltpu.SemaphoreType.DMA,
            pltpu.SemaphoreType.DMA,
            pltpu.SemaphoreType.DMA,
            pltpu.SemaphoreType.DMA,
            pltpu.SemaphoreType.DMA,
            pltpu.SemaphoreType.DMA,
            pltpu.SemaphoreType.DMA,
        ],
    )
    def k(idx_hbm, out_hbm, idxs, ob0, ob1, ob2, ob3, olds, si0, si1, si2, si3, so0, so1, so2, so3):
        outs = (ob0, ob1, ob2, ob3)
        wid = lax.axis_index("s") * 2 + lax.axis_index("c")
        base_chunk = wid * _CPW
        zeros16f = jnp.zeros((16,), jnp.float32)
        ones16f = jnp.ones((16,), jnp.float32)
        iota16 = lax.iota(jnp.int32, 16)
        isem = (si0, si1, si2, si3)
        osem = (so0, so1, so2, so3)

        for b in range(_NBUF):
            pltpu.async_copy(
                idx_hbm.at[pl.ds((base_chunk + b) * 1024, _W)],
                idxs.at[b], isem[b],
            )

        def zinit(t, _):
            for b in range(_NBUF):
                outs[b][pl.ds(t * 16, 16)] = zeros16f
            return 0

        lax.fori_loop(0, _OUTW // 16, zinit, 0, unroll=8)

        def cinit(g, _):
            init16 = g * 16 + iota16
            for b in range(_NBUF):
                olds[b, pl.ds(g * 16, 16)] = init16
            return 0

        lax.fori_loop(0, _W // 16, cinit, 0, unroll=8)

        def run_batch(o, b):
            p = o * _NBUF + b
            chunk = base_chunk + p
            word0 = chunk * 1024
            pltpu.make_async_copy(
                idx_hbm.at[pl.ds(word0, _W)], idxs.at[b], isem[b]
            ).wait()

            @pl.when(o >= 1)
            def _():
                pltpu.make_async_copy(
                    out_hbm.at[pl.ds(0, _OUTW)], outs[b], osem[b]
                ).wait()

            def group(g, _):
                base = g * 16
                idx16 = idxs[b, pl.ds(base, 16)]
                old16 = olds[b, pl.ds(base, 16)]
                plsc.store_scatter(outs[b], [old16], zeros16f)
                off16 = idx16 * _W + (base + iota16)
                olds[b, pl.ds(base, 16)] = off16
                plsc.store_scatter(outs[b], [off16], ones16f)
                return 0

            lax.fori_loop(0, _W // 16, group, 0, unroll=4)

            @pl.when(o < _CPW // _NBUF - 1)
            def _():
                pltpu.async_copy(
                    idx_hbm.at[pl.ds(word0 + _NBUF * 1024, _W)],
                    idxs.at[b], isem[b],
                )

            for c in range(_NC):
                pltpu.async_copy(
                    outs[b].at[pl.ds(c * _W, _W)],
                    out_hbm.at[pl.ds(c * _PLANE + word0, _W)],
                    osem[b],
                )
            return 0

        def outer(o, _):
            for b in range(_NBUF):
                run_batch(o, b)
            return 0

        lax.fori_loop(0, _CPW // _NBUF, outer, 0)

        for b in range(_NBUF):
            pltpu.make_async_copy(
                out_hbm.at[pl.ds(0, _OUTW)], outs[b], osem[b]
            ).wait()

    return k(idx_flat)


def kernel(inputs):
    t = (
        inputs.reshape(_IH, 8, _JH, 128)
        .transpose(0, 2, 1, 3)
        .reshape(_NCHUNK * 1024)
    )
    y = _sc_call(t)
    y5 = y.reshape(_NC, _IH, _JH, 8, 128)
    return y5.transpose(1, 3, 2, 4, 0).reshape(_B, _S, _NC)
